# Initial kernel scaffold; baseline (speedup 1.0000x reference)
import jax
import jax.numpy as jnp
from jax.experimental import pallas as pl


def kernel(x, edge_index, edge_attr, W1_rel, b1, W1_root, W2_rel, b2, W2_root, W3_rel, b3, W3_root):
    raise NotImplementedError("write your pallas kernel here")


# speedup vs baseline: 6.5592x; 6.5592x over previous
import functools
import jax
import jax.numpy as jnp
from jax import lax
from jax.experimental import pallas as pl
from jax.experimental.pallas import tpu as pltpu
from jax.experimental.pallas import tpu_sc as plsc

N = 50000
H = 64
NP = 51200
ROWS = 6400
EP = ROWS * 128
NPT = NP // 16


def _seg_scalar_call(src2d, dst2d, ew2d, vec_p, zeros1):
  mesh = plsc.VectorSubcoreMesh(core_axis_name="c", subcore_axis_name="s")
  rows_per_tile = ROWS // 32
  blk = 40

  @functools.partial(
      pl.kernel,
      mesh=mesh,
      compiler_params=pltpu.CompilerParams(needs_layout_passes=False),
      out_type=(
          jax.ShapeDtypeStruct((NP,), jnp.float32),
          jax.ShapeDtypeStruct((NP,), jnp.float32),
      ),
      scratch_types=[
          pltpu.VMEM((NP,), jnp.float32),
          pltpu.VMEM((blk, 128), jnp.int32),
          pltpu.VMEM((blk, 128), jnp.int32),
          pltpu.VMEM((blk, 128), jnp.float32),
          pltpu.VMEM((blk, 128), jnp.float32),
          pltpu.VMEM_SHARED((NP,), jnp.float32),
      ],
  )
  def k(src_h, dst_h, ew_h, vec_h, z_h, out0, out1, x_v, srcb, dstb, ewb,
        valsb, acc):
    c = lax.axis_index("c")
    s = lax.axis_index("s")
    w = c * 16 + s
    pltpu.sync_copy(z_h.at[pl.ds(s * NPT, NPT)], acc.at[pl.ds(s * NPT, NPT)])
    pltpu.sync_copy(vec_h, x_v)
    plsc.subcore_barrier()
    base = w * rows_per_tile
    for b in range(rows_per_tile // blk):
      r0 = base + b * blk
      pltpu.sync_copy(src_h.at[pl.ds(r0, blk)], srcb)
      pltpu.sync_copy(dst_h.at[pl.ds(r0, blk)], dstb)
      pltpu.sync_copy(ew_h.at[pl.ds(r0, blk)], ewb)

      def body(j, carry):
        for i in range(8):
          idx16 = srcb[j, pl.ds(i * 16, 16)]
          xv = plsc.load_gather(x_v, [idx16])
          valsb[j, pl.ds(i * 16, 16)] = xv * ewb[j, pl.ds(i * 16, 16)]
        pltpu.sync_copy(valsb.at[j], acc.at[dstb.at[j]], add=True)
        return carry

      lax.fori_loop(0, blk, body, 0)
    plsc.subcore_barrier()

    @pl.when(c == 0)
    def _():
      pltpu.sync_copy(acc.at[pl.ds(s * NPT, NPT)],
                      out0.at[pl.ds(s * NPT, NPT)])

    @pl.when(c == 1)
    def _():
      pltpu.sync_copy(acc.at[pl.ds(s * NPT, NPT)],
                      out1.at[pl.ds(s * NPT, NPT)])

  return k(src2d, dst2d, ew2d, vec_p, zeros1)


def _seg_rows_call(src2d, dst2d, ew2d, h1q, zeros16):
  mesh = plsc.VectorSubcoreMesh(core_axis_name="c", subcore_axis_name="s")
  rows_per_tile = ROWS // 16
  blk = 80

  @functools.partial(
      pl.kernel,
      mesh=mesh,
      compiler_params=pltpu.CompilerParams(needs_layout_passes=False,
                                           use_tc_tiling_on_sc=False),
      out_type=tuple(
          jax.ShapeDtypeStruct((NP, 16), jnp.float32) for _ in range(4)),
      scratch_types=[
          pltpu.VMEM((blk, 128), jnp.int32),
          pltpu.VMEM((blk, 128), jnp.int32),
          pltpu.VMEM((blk, 128), jnp.float32),
          pltpu.VMEM((128, 16), jnp.float32),
          pltpu.SemaphoreType.DMA,
          pltpu.VMEM_SHARED((NP, 16), jnp.float32),
      ],
  )
  def k(src_h, dst_h, ew_h, q0_h, q1_h, q2_h, q3_h, z_h, out0, out1, out2,
        out3, srcb, dstb, ewb, rowsb, gsem, acc):
    c = lax.axis_index("c")
    s = lax.axis_index("s")
    base = s * rows_per_tile
    sl = pl.ds(s * NPT, NPT)

    def quarter(table_h, out_h):
      pltpu.sync_copy(z_h.at[sl], acc.at[sl])
      plsc.subcore_barrier()
      for b in range(rows_per_tile // blk):
        r0 = base + b * blk
        pltpu.sync_copy(src_h.at[pl.ds(r0, blk)], srcb)
        pltpu.sync_copy(dst_h.at[pl.ds(r0, blk)], dstb)
        pltpu.sync_copy(ew_h.at[pl.ds(r0, blk)], ewb)

        def body(j, carry):
          pltpu.async_copy(table_h.at[srcb.at[j]], rowsb, gsem).wait()

          def mul(r, carry2):
            jv = jnp.full((16,), j, jnp.int32)
            rv = jnp.full((16,), r, jnp.int32)
            ws = plsc.load_gather(ewb, [jv, rv])
            rowsb[r, pl.ds(0, 16)] = rowsb[r, pl.ds(0, 16)] * ws
            return carry2

          lax.fori_loop(0, 128, mul, 0)
          pltpu.sync_copy(rowsb, acc.at[dstb.at[j]], add=True)
          return carry

        lax.fori_loop(0, blk, body, 0)
      plsc.subcore_barrier()
      pltpu.sync_copy(acc.at[sl], out_h.at[sl])
      plsc.subcore_barrier()

    @pl.when(c == 0)
    def _():
      quarter(q0_h, out0)
      quarter(q1_h, out1)

    @pl.when(c == 1)
    def _():
      quarter(q2_h, out2)
      quarter(q3_h, out3)

  return k(src2d, dst2d, ew2d, h1q[0], h1q[1], h1q[2], h1q[3], zeros16)


def _expand_body(a0, a1, xr, wrel, wroot, b1, o0, o1, o2, o3):
  a = a0[:] + a1[:]
  h = (a[:, None] * wrel[:][None, :] + xr[:][:, None] * wroot[:][None, :]
       + b1[:][None, :])
  h = jnp.maximum(h, 0.0)
  o0[:, :] = h[:, 0:16]
  o1[:, :] = h[:, 16:32]
  o2[:, :] = h[:, 32:48]
  o3[:, :] = h[:, 48:64]


def _expand_call(a0, a1, x1, wrel, wroot, b1):
  blk = 2048
  grid = NP // blk
  bs1 = pl.BlockSpec((blk,), lambda i: (i,))
  bsw = pl.BlockSpec((H,), lambda i: (0,))
  bso = pl.BlockSpec((blk, 16), lambda i: (i, 0))
  return pl.pallas_call(
      _expand_body,
      grid=(grid,),
      in_specs=[bs1, bs1, bs1, bsw, bsw, bsw],
      out_specs=[bso] * 4,
      out_shape=[jax.ShapeDtypeStruct((NP, 16), jnp.float32)] * 4,
  )(a0, a1, x1, wrel, wroot, b1)


_BLK2 = 2000


def _dense2_body(*refs):
  gq = refs[0:4]
  hq = refs[4:8]
  wr = refs[8:12]
  wt = refs[12:16]
  b2, w3rel, w3root, b3, po, qo = refs[16:]
  h2 = b2[:][None, :].astype(jnp.float32) * jnp.ones((_BLK2, 1), jnp.float32)
  for i in range(4):
    h2 = h2 + jnp.dot(gq[i][:, :], wr[i][:, :],
                      preferred_element_type=jnp.float32)
    h2 = h2 + jnp.dot(hq[i][:, :], wt[i][:, :],
                      preferred_element_type=jnp.float32)
  h2 = jnp.maximum(h2, 0.0)
  po[:, :] = jnp.dot(h2, w3rel[:, :], preferred_element_type=jnp.float32)
  qo[:, :] = (jnp.dot(h2, w3root[:, :], preferred_element_type=jnp.float32)
              + b3[0])


def _dense2_call(g2q, h1q, W2_rel, W2_root, b2, W3_rel, W3_root, b3):
  grid = N // _BLK2
  bsn = pl.BlockSpec((_BLK2, 16), lambda i: (i, 0))
  bsw = pl.BlockSpec((16, H), lambda i: (0, 0))
  bsb = pl.BlockSpec((H,), lambda i: (0,))
  bsw1 = pl.BlockSpec((H, 1), lambda i: (0, 0))
  bsb1 = pl.BlockSpec((1,), lambda i: (0,))
  bso = pl.BlockSpec((_BLK2, 1), lambda i: (i, 0))
  wrq = [W2_rel[16 * i:16 * (i + 1)] for i in range(4)]
  wtq = [W2_root[16 * i:16 * (i + 1)] for i in range(4)]
  return pl.pallas_call(
      _dense2_body,
      grid=(grid,),
      in_specs=[bsn] * 8 + [bsw] * 8 + [bsb, bsw1, bsw1, bsb1],
      out_specs=[bso, bso],
      out_shape=[jax.ShapeDtypeStruct((N, 1), jnp.float32)] * 2,
  )(*g2q, *h1q, *wrq, *wtq, b2, W3_rel, W3_root, b3)


def _combine_body(o0, o1, q, out):
  out[:, :] = o0[:, :] + o1[:, :] + q[:, :]


def _combine_call(o0, o1, q):
  blk = 2048
  grid = NP // blk
  bso = pl.BlockSpec((blk, 1), lambda i: (i, 0))
  return pl.pallas_call(
      _combine_body,
      grid=(grid,),
      in_specs=[bso, bso, bso],
      out_specs=bso,
      out_shape=jax.ShapeDtypeStruct((NP, 1), jnp.float32),
  )(o0, o1, q)


def kernel(x, edge_index, edge_attr, W1_rel, b1, W1_root, W2_rel, b2,
           W2_root, W3_rel, b3, W3_root):
  x1 = x[:, 0].astype(jnp.float32)
  ew = edge_attr.astype(jnp.float32)
  E = edge_index.shape[1]
  pad = EP - E
  src2d = jnp.concatenate(
      [edge_index[0], jnp.zeros((pad,), jnp.int32)]).reshape(ROWS, 128)
  dst2d = jnp.concatenate(
      [edge_index[1], jnp.zeros((pad,), jnp.int32)]).reshape(ROWS, 128)
  ew2d = jnp.concatenate(
      [ew, jnp.zeros((pad,), jnp.float32)]).reshape(ROWS, 128)
  zeros1 = jnp.zeros((NP,), jnp.float32)
  zeros16 = jnp.zeros((NP, 16), jnp.float32)
  xp = jnp.concatenate([x1, jnp.zeros((NP - N,), jnp.float32)])

  a0, a1 = _seg_scalar_call(src2d, dst2d, ew2d, xp, zeros1)
  h1q = _expand_call(a0, a1, xp, W1_rel[0], W1_root[0], b1)

  g2q = _seg_rows_call(src2d, dst2d, ew2d, h1q, zeros16)
  g2q = [g[:N] for g in g2q]
  h1qn = [h[:N] for h in h1q]

  p, q = _dense2_call(g2q, h1qn, W2_rel, W2_root, b2, W3_rel, W3_root, b3)

  pp = jnp.concatenate([p[:, 0], jnp.zeros((NP - N,), jnp.float32)])
  o0, o1 = _seg_scalar_call(src2d, dst2d, ew2d, pp, zeros1)
  qp = jnp.concatenate([q, jnp.zeros((NP - N, 1), jnp.float32)], axis=0)
  out = _combine_call(o0[:, None], o1[:, None], qp)
  return out[:N]
